# baseline (speedup 1.0000x reference)
import jax
import jax.numpy as jnp
from jax import lax
from jax.experimental import pallas as pl
from jax.experimental.pallas import tpu as pltpu
from jax.experimental.pallas import tpu_sc as plsc

VOCAB = 100000
BATCH = 4096
HIST = 50

NC = 2
NS = 16
NW = NC * NS
COLS_PER_W = BATCH // NW
CHUNK = 5


def _body(idx_hbm, table_hbm, out_hbm, table_sh, idx_v, out_v, sem, isem):
    cid = lax.axis_index("c")
    sid = lax.axis_index("s")
    wid = sid * NC + cid
    c0 = wid * COLS_PER_W

    icp = pltpu.async_copy(idx_hbm.at[:, pl.ds(c0, COLS_PER_W)], idx_v, isem)

    @pl.when(sid == 0)
    def _():
        pltpu.sync_copy(table_hbm, table_sh)

    plsc.subcore_barrier()
    icp.wait()

    def step(c, carry):
        row = c * CHUNK
        cps = [
            pltpu.async_copy(
                table_sh.at[idx_v.at[row + j]], out_v.at[row + j], sem
            )
            for j in range(CHUNK)
        ]
        for cp in cps:
            cp.wait()
        return carry

    lax.fori_loop(0, HIST // CHUNK, step, 0)
    pltpu.sync_copy(out_v, out_hbm.at[:, pl.ds(c0, COLS_PER_W)])


@jax.jit
def kernel(inputs, table):
    mesh = plsc.VectorSubcoreMesh(core_axis_name="c", subcore_axis_name="s")
    out_t = pl.kernel(
        _body,
        out_type=jax.ShapeDtypeStruct((HIST, BATCH), jnp.float32),
        mesh=mesh,
        compiler_params=pltpu.CompilerParams(needs_layout_passes=False),
        scratch_types=[
            pltpu.VMEM_SHARED((VOCAB,), jnp.float32),
            pltpu.VMEM((HIST, COLS_PER_W), jnp.int32),
            pltpu.VMEM((HIST, COLS_PER_W), jnp.float32),
            pltpu.SemaphoreType.DMA,
            pltpu.SemaphoreType.DMA,
        ],
    )(inputs.T, table)
    return out_t.T


# speedup vs baseline: 1.0677x; 1.0677x over previous
import jax
import jax.numpy as jnp
from jax import lax
from jax.experimental import pallas as pl
from jax.experimental.pallas import tpu as pltpu
from jax.experimental.pallas import tpu_sc as plsc

VOCAB = 100000
BATCH = 4096
HIST = 50

NC = 2
NS = 16
NW = NC * NS
COLS_PER_W = BATCH // NW
CHUNK = 25


def _body(idx_hbm, table_hbm, out_hbm, table_sh, idx_v, out_v, sem, isem):
    cid = lax.axis_index("c")
    sid = lax.axis_index("s")
    wid = sid * NC + cid
    c0 = wid * COLS_PER_W

    icp = pltpu.async_copy(idx_hbm.at[:, pl.ds(c0, COLS_PER_W)], idx_v, isem)

    @pl.when(sid == 0)
    def _():
        pltpu.sync_copy(table_hbm, table_sh)

    plsc.subcore_barrier()
    icp.wait()

    def step(c, carry):
        row = c * CHUNK
        cps = [
            pltpu.async_copy(
                table_sh.at[idx_v.at[row + j]], out_v.at[row + j], sem
            )
            for j in range(CHUNK)
        ]
        for cp in cps:
            cp.wait()
        return carry

    lax.fori_loop(0, HIST // CHUNK, step, 0)
    pltpu.sync_copy(out_v, out_hbm.at[:, pl.ds(c0, COLS_PER_W)])


@jax.jit
def kernel(inputs, table):
    mesh = plsc.VectorSubcoreMesh(core_axis_name="c", subcore_axis_name="s")
    out_t = pl.kernel(
        _body,
        out_type=jax.ShapeDtypeStruct((HIST, BATCH), jnp.float32),
        mesh=mesh,
        compiler_params=pltpu.CompilerParams(needs_layout_passes=False),
        scratch_types=[
            pltpu.VMEM_SHARED((VOCAB,), jnp.float32),
            pltpu.VMEM((HIST, COLS_PER_W), jnp.int32),
            pltpu.VMEM((HIST, COLS_PER_W), jnp.float32),
            pltpu.SemaphoreType.DMA,
            pltpu.SemaphoreType.DMA,
        ],
    )(inputs.T, table)
    return out_t.T
